# baseline (device time: 75410 ns/iter reference)
import jax
import jax.numpy as jnp
from jax import lax
from jax.experimental import pallas as pl
from jax.experimental.pallas import tpu as pltpu

N_DEV = 4


def _ring_allreduce_bidir(p):
    rows, cols = p.shape
    half = rows // 2
    chunk = half // N_DEV

    def body(p_ref, out_ref, r_buf, st_buf, a_buf, send_sems, recv_sems):
        my = lax.axis_index("i")
        left = lax.rem(my + N_DEV - 1, N_DEV)
        right = lax.rem(my + 1, N_DEV)

        barrier_sem = pltpu.get_barrier_semaphore()
        for nbr in (left, right):
            pl.semaphore_signal(
                barrier_sem, inc=1,
                device_id=(nbr,), device_id_type=pl.DeviceIdType.MESH,
            )
        pl.semaphore_wait(barrier_sem, 2)

        peer = (right, left)
        base = (0, half)

        def crow(d, idx):
            return base[d] + lax.rem(idx + 4 * N_DEV, N_DEV) * chunk

        rdmas = []

        for s in range(N_DEV - 1):
            step_rdmas = []
            for d in range(2):
                sgn = 1 if d == 0 else -1
                if s == 0:
                    src = p_ref.at[pl.ds(crow(d, my), chunk)]
                else:
                    src = st_buf.at[d, s - 1]
                rdma = pltpu.make_async_remote_copy(
                    src_ref=src,
                    dst_ref=r_buf.at[d, s],
                    send_sem=send_sems.at[d, s],
                    recv_sem=recv_sems.at[d, s],
                    device_id=(peer[d],),
                    device_id_type=pl.DeviceIdType.MESH,
                )
                rdma.start()
                step_rdmas.append(rdma)
            for d in range(2):
                sgn = 1 if d == 0 else -1
                step_rdmas[d].wait_recv()
                st_buf[d, s] = r_buf[d, s] + p_ref[
                    pl.ds(crow(d, my - sgn * (s + 1)), chunk)
                ]
            rdmas.extend(step_rdmas)

        for d in range(2):
            sgn = 1 if d == 0 else -1
            out_ref[pl.ds(crow(d, my + sgn), chunk), :] = st_buf[
                d, N_DEV - 2
            ].astype(jnp.float32)

        for s in range(N_DEV - 1):
            step_rdmas = []
            for d in range(2):
                src = st_buf.at[d, N_DEV - 2] if s == 0 else a_buf.at[d, s - 1]
                rdma = pltpu.make_async_remote_copy(
                    src_ref=src,
                    dst_ref=a_buf.at[d, s],
                    send_sem=send_sems.at[d, N_DEV - 1 + s],
                    recv_sem=recv_sems.at[d, N_DEV - 1 + s],
                    device_id=(peer[d],),
                    device_id_type=pl.DeviceIdType.MESH,
                )
                rdma.start()
                step_rdmas.append(rdma)
            for d in range(2):
                sgn = 1 if d == 0 else -1
                step_rdmas[d].wait_recv()
                out_ref[pl.ds(crow(d, my - sgn * s), chunk), :] = a_buf[
                    d, s
                ].astype(jnp.float32)
            rdmas.extend(step_rdmas)

        for rdma in rdmas:
            rdma.wait_send()

    n_step = 2 * (N_DEV - 1)
    return pl.pallas_call(
        body,
        out_shape=jax.ShapeDtypeStruct((rows, cols), jnp.float32),
        in_specs=[pl.BlockSpec(memory_space=pltpu.VMEM)],
        out_specs=pl.BlockSpec(memory_space=pltpu.VMEM),
        scratch_shapes=[
            pltpu.VMEM((2, N_DEV - 1, chunk, cols), p.dtype),
            pltpu.VMEM((2, N_DEV - 1, chunk, cols), p.dtype),
            pltpu.VMEM((2, N_DEV - 1, chunk, cols), p.dtype),
            pltpu.SemaphoreType.DMA((2, n_step)),
            pltpu.SemaphoreType.DMA((2, n_step)),
        ],
        compiler_params=pltpu.CompilerParams(collective_id=0),
    )(p)


def _attn_partial_pallas(xb, Wqb, K2, V2, Wob, R, Dh, scale):
    B, Sq, Dm = xb.shape
    G, Skv = K2.shape[1], K2.shape[2]
    RDh = R * Dh

    def body(x_ref, wq_ref, k_ref, v_ref, wo_ref, o_ref, acc_ref):
        g = pl.program_id(1)
        xblk = x_ref[0]
        q = (
            jnp.dot(xblk, wq_ref[...], preferred_element_type=jnp.float32)
            * scale
        ).astype(jnp.bfloat16)
        k = k_ref[0, 0]
        v = v_ref[0, 0]
        heads = []
        for r in range(R):
            qr = q[:, r * Dh:(r + 1) * Dh]
            s = lax.dot_general(
                qr, k, (((1,), (1,)), ((), ())),
                preferred_element_type=jnp.float32,
            )
            m = jnp.max(s, axis=1, keepdims=True)
            p = jnp.exp(s - m)
            l = jnp.sum(p, axis=1, keepdims=True)
            o = jnp.dot(
                p.astype(jnp.bfloat16), v,
                preferred_element_type=jnp.float32,
            ) / l
            heads.append(o.astype(jnp.bfloat16))
        attn_blk = jnp.concatenate(heads, axis=1)
        contrib = jnp.dot(
            attn_blk, wo_ref[...], preferred_element_type=jnp.float32
        )

        @pl.when(g == 0)
        def _():
            acc_ref[...] = contrib

        @pl.when(g > 0)
        def _():
            acc_ref[...] += contrib

        @pl.when(g == G - 1)
        def _():
            o_ref[0] = acc_ref[...].astype(jnp.bfloat16)

    return pl.pallas_call(
        body,
        grid=(B, G),
        in_specs=[
            pl.BlockSpec((1, Sq, Dm), lambda b, g: (b, 0, 0)),
            pl.BlockSpec((Dm, RDh), lambda b, g: (0, g)),
            pl.BlockSpec((1, 1, Skv, Dh), lambda b, g: (b, g, 0, 0)),
            pl.BlockSpec((1, 1, Skv, Dh), lambda b, g: (b, g, 0, 0)),
            pl.BlockSpec((RDh, Dm), lambda b, g: (g, 0)),
        ],
        out_specs=pl.BlockSpec((1, Sq, Dm), lambda b, g: (b, 0, 0)),
        out_shape=jax.ShapeDtypeStruct((B, Sq, Dm), jnp.bfloat16),
        scratch_shapes=[pltpu.VMEM((Sq, Dm), jnp.float32)],
    )(xb, Wqb, K2, V2, Wob)


def kernel(x, Wq, Wo, K_ext, V_ext):
    B, Sq, Dm = x.shape
    Dh = 128
    Hq_local = Wq.shape[1] // Dh
    G = 2
    R = Hq_local // G
    scale = 0.08838834764831843

    i = lax.axis_index("i")

    xb = x.astype(jnp.bfloat16)
    Wqb = Wq.astype(jnp.bfloat16)
    Wob = Wo.astype(jnp.bfloat16)

    K2 = lax.dynamic_slice_in_dim(K_ext, G * i, G, axis=2)
    V2 = lax.dynamic_slice_in_dim(V_ext, G * i, G, axis=2)
    K2 = K2.transpose(0, 2, 1, 3).astype(jnp.bfloat16)
    V2 = V2.transpose(0, 2, 1, 3).astype(jnp.bfloat16)

    partial = _attn_partial_pallas(xb, Wqb, K2, V2, Wob, R, Dh, scale)

    out2d = _ring_allreduce_bidir(partial.reshape(B * Sq, Dm))
    return out2d.reshape(B, Sq, Dm)
